# Initial kernel scaffold; baseline (speedup 1.0000x reference)
import jax
import jax.numpy as jnp
from jax.experimental import pallas as pl


def kernel(nodes, adj, emb, W, b):
    raise NotImplementedError("write your pallas kernel here")


# speedup vs baseline: 1.2469x; 1.2469x over previous
import functools

import jax
import jax.numpy as jnp
from jax import lax
from jax.experimental import pallas as pl
from jax.experimental.pallas import tpu as pltpu
from jax.experimental.pallas import tpu_sc as plsc

N_NODES = 10000
DEG = 16
D = 256
B = 10000
BP = 10240
NC = 2
NS = 16
NW = NC * NS
BPW = BP // NW
CH = 8
NCHUNK = BPW // CH


def _sc_gather_kernel(nodes_h, adj_h, emb_h, self_h, nsum_h,
                      idxc, adjc, nidx, sbuf, nbuf, mbuf):
    cid = lax.axis_index("c")
    sid = lax.axis_index("s")
    wid = sid * NC + cid
    base = wid * BPW

    pltpu.sync_copy(nodes_h.at[pl.ds(wid * NCHUNK, NCHUNK)], idxc)

    def chunk_body(c, carry):
        ids = idxc.at[c]
        pltpu.sync_copy(adj_h.at[ids], adjc)
        for r in range(CH):
            nidx[pl.ds(r * DEG, DEG)] = adjc[r, pl.ds(0, DEG)]
        pltpu.sync_copy(emb_h.at[ids], sbuf)
        pltpu.sync_copy(emb_h.at[nidx], nbuf)
        for r in range(CH):
            for v in range(D // 16):
                cs = pl.ds(v * 16, 16)
                a = nbuf[r * DEG, cs]
                for j in range(1, DEG):
                    a = a + nbuf[r * DEG + j, cs]
                mbuf[r, cs] = a
        pltpu.sync_copy(sbuf, self_h.at[pl.ds(base + c * CH, CH)])
        pltpu.sync_copy(mbuf, nsum_h.at[pl.ds(base + c * CH, CH)])
        return carry

    lax.fori_loop(0, NCHUNK, chunk_body, 0)


def _sc_gather(nodes_p, adj, emb):
    mesh = plsc.VectorSubcoreMesh(core_axis_name="c", subcore_axis_name="s")
    kern = functools.partial(
        pl.kernel,
        mesh=mesh,
        out_type=(
            jax.ShapeDtypeStruct((BP, D), jnp.float32),
            jax.ShapeDtypeStruct((BP, D), jnp.float32),
        ),
        scratch_types=[
            pltpu.VMEM((NCHUNK, CH), jnp.int32),
            pltpu.VMEM((CH, 128), jnp.int32),
            pltpu.VMEM((CH * DEG,), jnp.int32),
            pltpu.VMEM((CH, D), jnp.float32),
            pltpu.VMEM((CH * DEG, D), jnp.float32),
            pltpu.VMEM((CH, D), jnp.float32),
        ],
    )(_sc_gather_kernel)
    return kern(nodes_p, adj, emb)


def _mm_kernel(x1_ref, x2_ref, w1_ref, w2_ref, b_ref, o_ref):
    acc = jnp.dot(x1_ref[...], w1_ref[...], preferred_element_type=jnp.float32)
    acc = acc + jnp.dot(x2_ref[...], w2_ref[...], preferred_element_type=jnp.float32)
    o_ref[...] = jnp.maximum(acc + b_ref[...], 0.0)


def _tc_matmul(self_f, nsum, w1, w2, b2):
    bm = 1024
    grid = (BP // bm,)
    return pl.pallas_call(
        _mm_kernel,
        grid=grid,
        in_specs=[
            pl.BlockSpec((bm, D), lambda i: (i, 0)),
            pl.BlockSpec((bm, D), lambda i: (i, 0)),
            pl.BlockSpec((D, D), lambda i: (0, 0)),
            pl.BlockSpec((D, D), lambda i: (0, 0)),
            pl.BlockSpec((1, D), lambda i: (0, 0)),
        ],
        out_specs=pl.BlockSpec((bm, D), lambda i: (i, 0)),
        out_shape=jax.ShapeDtypeStruct((BP, D), jnp.float32),
    )(self_f, nsum, w1, w2, b2)


def kernel(nodes, adj, emb, W, b):
    nodes_p = jnp.pad(nodes.astype(jnp.int32), (0, BP - B)).reshape(NW * NCHUNK, CH)
    adj_p = jnp.pad(adj.astype(jnp.int32), ((0, 0), (0, 128 - DEG)))
    self_f, nsum = _sc_gather(nodes_p, adj_p, emb)
    w1 = W[:D, :]
    w2 = W[D:, :] * (1.0 / DEG)
    out = _tc_matmul(self_f, nsum, w1, w2, b.reshape(1, D))
    return out[:B]


# speedup vs baseline: 3.0144x; 2.4175x over previous
import functools

import jax
import jax.numpy as jnp
from jax import lax
from jax.experimental import pallas as pl
from jax.experimental.pallas import tpu as pltpu
from jax.experimental.pallas import tpu_sc as plsc

N_NODES = 10000
DEG = 16
D = 256
B = 10000
BP = 10240
NC = 2
NS = 16
NW = NC * NS
BPW = BP // NW
CH = 8
NCHUNK = BPW // CH


def _sc_gather_kernel(nodes_h, adj_h, emb_h, self_h, nsum_h,
                      idxc, adjc0, adjc1, nidx_all,
                      sbuf0, sbuf1, nbuf0, nbuf1, mbuf0, mbuf1,
                      sga0, sga1, sgn0, sgn1, sgs0, sgs1,
                      swm0, swm1, sws0, sws1):
    cid = lax.axis_index("c")
    sid = lax.axis_index("s")
    wid = sid * NC + cid
    base = wid * BPW

    pltpu.sync_copy(nodes_h.at[pl.ds(wid * NCHUNK, NCHUNK)], idxc)

    def adj_dma(c, buf, sem):
        return pltpu.make_async_copy(adj_h.at[idxc.at[c]], buf, sem)

    def stage_nidx(c, buf):
        for r in range(CH):
            nidx_all[c, pl.ds(r * DEG, DEG)] = buf[r, pl.ds(0, DEG)]

    adj_dma(0, adjc0, sga0).start()

    def phase1(i, carry):
        c = i * 2
        adj_dma(c + 1, adjc1, sga1).start()
        adj_dma(c, adjc0, sga0).wait()
        stage_nidx(c, adjc0)

        @pl.when(c + 2 < NCHUNK)
        def _():
            adj_dma(c + 2, adjc0, sga0).start()

        adj_dma(c + 1, adjc1, sga1).wait()
        stage_nidx(c + 1, adjc1)
        return carry

    lax.fori_loop(0, NCHUNK // 2, phase1, 0)

    def n_dma(c, nb, sem):
        return pltpu.make_async_copy(emb_h.at[nidx_all.at[c]], nb, sem)

    def s_dma(c, sb, sem):
        return pltpu.make_async_copy(emb_h.at[idxc.at[c]], sb, sem)

    def wm_dma(c, mb, sem):
        return pltpu.make_async_copy(mb, nsum_h.at[pl.ds(base + c * CH, CH)], sem)

    def ws_dma(c, sb, sem):
        return pltpu.make_async_copy(sb, self_h.at[pl.ds(base + c * CH, CH)], sem)

    def reduce_chunk(nb, mb):
        def row(r, carry):
            for v in range(D // 16):
                cs = pl.ds(v * 16, 16)
                a = nb[r * DEG, cs]
                for j in range(1, DEG):
                    a = a + nb[r * DEG + j, cs]
                mb[r, cs] = a
            return carry
        lax.fori_loop(0, CH, row, 0)

    n_dma(0, nbuf0, sgn0).start()
    s_dma(0, sbuf0, sgs0).start()

    def half(c, nb, sb, mb, sgn, sgs, swm, sws, nb_n, sb_n, sgn_n, sgs_n,
             sws_n):
        @pl.when(c + 1 < NCHUNK)
        def _():
            @pl.when(c >= 1)
            def _():
                ws_dma(c - 1, sb_n, sws_n).wait()
            n_dma(c + 1, nb_n, sgn_n).start()
            s_dma(c + 1, sb_n, sgs_n).start()

        n_dma(c, nb, sgn).wait()
        s_dma(c, sb, sgs).wait()
        ws_dma(c, sb, sws).start()

        @pl.when(c >= 2)
        def _():
            wm_dma(c - 2, mb, swm).wait()
        reduce_chunk(nb, mb)
        wm_dma(c, mb, swm).start()

    def phase2(i, carry):
        c = i * 2
        half(c, nbuf0, sbuf0, mbuf0, sgn0, sgs0, swm0, sws0,
             nbuf1, sbuf1, sgn1, sgs1, sws1)
        half(c + 1, nbuf1, sbuf1, mbuf1, sgn1, sgs1, swm1, sws1,
             nbuf0, sbuf0, sgn0, sgs0, sws0)
        return carry

    lax.fori_loop(0, NCHUNK // 2, phase2, 0)

    wm_dma(NCHUNK - 2, mbuf0, swm0).wait()
    ws_dma(NCHUNK - 2, sbuf0, sws0).wait()
    wm_dma(NCHUNK - 1, mbuf1, swm1).wait()
    ws_dma(NCHUNK - 1, sbuf1, sws1).wait()


def _sc_gather(nodes_p, adj_p, emb):
    mesh = plsc.VectorSubcoreMesh(core_axis_name="c", subcore_axis_name="s")
    kern = functools.partial(
        pl.kernel,
        mesh=mesh,
        out_type=(
            jax.ShapeDtypeStruct((BP, D), jnp.float32),
            jax.ShapeDtypeStruct((BP, D), jnp.float32),
        ),
        scratch_types=[
            pltpu.VMEM((NCHUNK, CH), jnp.int32),
            pltpu.VMEM((CH, 128), jnp.int32),
            pltpu.VMEM((CH, 128), jnp.int32),
            pltpu.VMEM((NCHUNK, CH * DEG), jnp.int32),
            pltpu.VMEM((CH, D), jnp.float32),
            pltpu.VMEM((CH, D), jnp.float32),
            pltpu.VMEM((CH * DEG, D), jnp.float32),
            pltpu.VMEM((CH * DEG, D), jnp.float32),
            pltpu.VMEM((CH, D), jnp.float32),
            pltpu.VMEM((CH, D), jnp.float32),
        ] + [pltpu.SemaphoreType.DMA] * 10,
    )(_sc_gather_kernel)
    return kern(nodes_p, adj_p, emb)


def _mm_kernel(x1_ref, x2_ref, w1_ref, w2_ref, b_ref, o_ref):
    acc = jnp.dot(x1_ref[...], w1_ref[...], preferred_element_type=jnp.float32)
    acc = acc + jnp.dot(x2_ref[...], w2_ref[...], preferred_element_type=jnp.float32)
    o_ref[...] = jnp.maximum(acc + b_ref[...], 0.0)


def _tc_matmul(self_f, nsum, w1, w2, b2):
    bm = 1000
    grid = (B // bm,)
    return pl.pallas_call(
        _mm_kernel,
        grid=grid,
        in_specs=[
            pl.BlockSpec((bm, D), lambda i: (i, 0)),
            pl.BlockSpec((bm, D), lambda i: (i, 0)),
            pl.BlockSpec((D, D), lambda i: (0, 0)),
            pl.BlockSpec((D, D), lambda i: (0, 0)),
            pl.BlockSpec((1, D), lambda i: (0, 0)),
        ],
        out_specs=pl.BlockSpec((bm, D), lambda i: (i, 0)),
        out_shape=jax.ShapeDtypeStruct((B, D), jnp.float32),
    )(self_f, nsum, w1, w2, b2)


def kernel(nodes, adj, emb, W, b):
    nodes_p = jnp.pad(nodes.astype(jnp.int32), (0, BP - B)).reshape(NW * NCHUNK, CH)
    adj_p = jnp.pad(adj.astype(jnp.int32), ((0, 0), (0, 128 - DEG)))
    self_f, nsum = _sc_gather(nodes_p, adj_p, emb)
    w1 = W[:D, :]
    w2 = W[D:, :] * (1.0 / DEG)
    return _tc_matmul(self_f, nsum, w1, w2, b.reshape(1, D))
